# Initial kernel scaffold; baseline (speedup 1.0000x reference)
import jax
import jax.numpy as jnp
from jax.experimental import pallas as pl


def kernel(grid_ids, grid_types, row_positions, col_positions, grid_id_table, grid_type_table, row_table, col_table, input_weight, position_weight, combine_weights):
    raise NotImplementedError("write your pallas kernel here")


# speedup vs baseline: 2.9846x; 2.9846x over previous
import functools

import jax
import jax.numpy as jnp
from jax import lax
from jax.experimental import pallas as pl
from jax.experimental.pallas import tpu as pltpu
from jax.experimental.pallas import tpu_sc as plsc

_D = 1024
_DH = _D // 2
_B, _S = 4, 8192
_NTOK = _B * _S
_ROWS = 2 * _NTOK
_NC, _NS = 2, 16
_NW = _NC * _NS
_RPW = _ROWS // _NW
_CH = 128
_NITER = _RPW // _CH


def _prep_body(gid_tab, gty_tab, row_tab, col_tab, iw, pw, cw,
               gids, gtys, rpos, cpos, h1, h2, i1, i2):
    c = cw[...]
    e = jnp.exp(c - jnp.max(c))
    w = e / jnp.sum(e)
    w0 = w[0, 0]
    w1 = w[0, 1]
    a = w0 * iw[0, 0]
    b = w0 * pw[0, 0]

    def _l2n(x):
        n = jnp.sqrt(jnp.sum(x * x, axis=-1, keepdims=True))
        return x / jnp.maximum(n, 1e-12)

    gin = a * _l2n(gid_tab[...])
    gtn = b * _l2n(gty_tab[...])
    row = w1 * row_tab[...]
    col = w1 * col_tab[...]

    h1[...] = (gin[:, None, None, :_DH] + gtn[None, :, None, :_DH]
               + row[None, None, :, :])
    h2[...] = (gin[:, None, None, _DH:] + gtn[None, :, None, _DH:]
               + col[None, None, :, :])

    base = (gids[...] * 2 + gtys[...]) * 30
    i1[...] = base + rpos[...]
    i2[...] = 600 + base + cpos[...]


_prep = pl.pallas_call(
    _prep_body,
    out_shape=[
        jax.ShapeDtypeStruct((10, 2, 30, _DH), jnp.float32),
        jax.ShapeDtypeStruct((10, 2, 30, _DH), jnp.float32),
        jax.ShapeDtypeStruct((_B, _S), jnp.int32),
        jax.ShapeDtypeStruct((_B, _S), jnp.int32),
    ],
)


def _sc_body(h_hbm, idx_hbm, out_hbm, idx_v, rows_v, sem):
    wid = lax.axis_index("s") * _NC + lax.axis_index("c")
    base = wid * _RPW

    def step(i, carry):
        off = base + i * _CH
        pltpu.sync_copy(idx_hbm.at[pl.ds(off, _CH)], idx_v)
        pltpu.async_copy(h_hbm.at[idx_v], rows_v, sem).wait()
        pltpu.sync_copy(rows_v, out_hbm.at[pl.ds(off, _CH)])
        return carry

    lax.fori_loop(0, _NITER, step, 0)


@functools.cache
def _sc_gather():
    return pl.kernel(
        _sc_body,
        out_type=jax.ShapeDtypeStruct((_ROWS, _DH), jnp.float32),
        mesh=plsc.VectorSubcoreMesh(core_axis_name="c", subcore_axis_name="s",
                                    num_cores=_NC, num_subcores=_NS),
        scratch_types=[
            pltpu.VMEM((_CH,), jnp.int32),
            pltpu.VMEM((_CH, _DH), jnp.float32),
            pltpu.SemaphoreType.DMA,
        ],
    )


def kernel(grid_ids, grid_types, row_positions, col_positions,
           grid_id_table, grid_type_table, row_table, col_table,
           input_weight, position_weight, combine_weights):
    gids = grid_ids.astype(jnp.int32)
    gtys = grid_types.astype(jnp.int32)
    rpos = row_positions.astype(jnp.int32)
    cpos = col_positions.astype(jnp.int32)

    h1, h2, i1, i2 = _prep(
        grid_id_table, grid_type_table, row_table, col_table,
        input_weight.reshape(1, 1), position_weight.reshape(1, 1),
        combine_weights.reshape(1, 4), gids, gtys, rpos, cpos)

    htab = jnp.concatenate(
        [h1.reshape(600, _DH), h2.reshape(600, _DH)], axis=0)
    idx = jnp.stack(
        [i1.reshape(-1), i2.reshape(-1)], axis=-1).reshape(-1)

    out = _sc_gather()(htab, idx)
    return out.reshape(_B, _S, _D)


# speedup vs baseline: 3.0020x; 1.0058x over previous
import functools

import jax
import jax.numpy as jnp
from jax import lax
from jax.experimental import pallas as pl
from jax.experimental.pallas import tpu as pltpu
from jax.experimental.pallas import tpu_sc as plsc

_D = 1024
_DH = _D // 2
_B, _S = 4, 8192
_NTOK = _B * _S
_ROWS = 2 * _NTOK
_NC, _NS = 2, 16
_NW = _NC * _NS
_RPW = _ROWS // _NW
_CH = 64
_NITER = _RPW // _CH


def _prep_body(gid_tab, gty_tab, row_tab, col_tab, iw, pw, cw,
               gids, gtys, rpos, cpos, h1, h2, i1, i2):
    c = cw[...]
    e = jnp.exp(c - jnp.max(c))
    w = e / jnp.sum(e)
    w0 = w[0, 0]
    w1 = w[0, 1]
    a = w0 * iw[0, 0]
    b = w0 * pw[0, 0]

    def _l2n(x):
        n = jnp.sqrt(jnp.sum(x * x, axis=-1, keepdims=True))
        return x / jnp.maximum(n, 1e-12)

    gin = a * _l2n(gid_tab[...])
    gtn = b * _l2n(gty_tab[...])
    row = w1 * row_tab[...]
    col = w1 * col_tab[...]

    h1[...] = (gin[:, None, None, :_DH] + gtn[None, :, None, :_DH]
               + row[None, None, :, :])
    h2[...] = (gin[:, None, None, _DH:] + gtn[None, :, None, _DH:]
               + col[None, None, :, :])

    base = (gids[...] * 2 + gtys[...]) * 30
    i1[...] = base + rpos[...]
    i2[...] = 600 + base + cpos[...]


_prep = pl.pallas_call(
    _prep_body,
    out_shape=[
        jax.ShapeDtypeStruct((10, 2, 30, _DH), jnp.float32),
        jax.ShapeDtypeStruct((10, 2, 30, _DH), jnp.float32),
        jax.ShapeDtypeStruct((_B, _S), jnp.int32),
        jax.ShapeDtypeStruct((_B, _S), jnp.int32),
    ],
)


def _sc_body(h_hbm, idx_hbm, out_hbm, idx_all, rows0, rows1,
             sg0, sg1, so0, so1):
    wid = lax.axis_index("s") * _NC + lax.axis_index("c")
    base = wid * _RPW

    pltpu.sync_copy(idx_hbm.at[pl.ds(base, _RPW)], idx_all)

    def _gsrc(i):
        return h_hbm.at[idx_all.at[pl.ds(i * _CH, _CH)]]

    def _dst(i):
        return out_hbm.at[pl.ds(base + i * _CH, _CH)]

    def _gstart(i, rows, sem):
        pltpu.async_copy(_gsrc(i), rows, sem)

    def _gwait(i, rows, sem):
        pltpu.make_async_copy(_gsrc(i), rows, sem).wait()

    def _wstart(i, rows, sem):
        pltpu.async_copy(rows, _dst(i), sem)

    def _wwait(i, rows, sem):
        pltpu.make_async_copy(rows, _dst(i), sem).wait()

    _gstart(0, rows0, sg0)
    _gstart(1, rows1, sg1)

    def step(j, carry):
        i0 = 2 * j
        i1 = i0 + 1
        _gwait(i0, rows0, sg0)
        _wstart(i0, rows0, so0)
        _gwait(i1, rows1, sg1)
        _wstart(i1, rows1, so1)

        @pl.when(j < _NITER // 2 - 1)
        def _():
            _wwait(i0, rows0, so0)
            _gstart(i0 + 2, rows0, sg0)
            _wwait(i1, rows1, so1)
            _gstart(i1 + 2, rows1, sg1)

        return carry

    lax.fori_loop(0, _NITER // 2, step, 0)
    _wwait(_NITER - 2, rows0, so0)
    _wwait(_NITER - 1, rows1, so1)


@functools.cache
def _sc_gather():
    return pl.kernel(
        _sc_body,
        out_type=jax.ShapeDtypeStruct((_ROWS, _DH), jnp.float32),
        mesh=plsc.VectorSubcoreMesh(core_axis_name="c", subcore_axis_name="s",
                                    num_cores=_NC, num_subcores=_NS),
        scratch_types=[
            pltpu.VMEM((_RPW,), jnp.int32),
            pltpu.VMEM((_CH, _DH), jnp.float32),
            pltpu.VMEM((_CH, _DH), jnp.float32),
            pltpu.SemaphoreType.DMA,
            pltpu.SemaphoreType.DMA,
            pltpu.SemaphoreType.DMA,
            pltpu.SemaphoreType.DMA,
        ],
    )


def kernel(grid_ids, grid_types, row_positions, col_positions,
           grid_id_table, grid_type_table, row_table, col_table,
           input_weight, position_weight, combine_weights):
    gids = grid_ids.astype(jnp.int32)
    gtys = grid_types.astype(jnp.int32)
    rpos = row_positions.astype(jnp.int32)
    cpos = col_positions.astype(jnp.int32)

    h1, h2, i1, i2 = _prep(
        grid_id_table, grid_type_table, row_table, col_table,
        input_weight.reshape(1, 1), position_weight.reshape(1, 1),
        combine_weights.reshape(1, 4), gids, gtys, rpos, cpos)

    htab = jnp.concatenate(
        [h1.reshape(600, _DH), h2.reshape(600, _DH)], axis=0)
    idx = jnp.stack(
        [i1.reshape(-1), i2.reshape(-1)], axis=-1).reshape(-1)

    out = _sc_gather()(htab, idx)
    return out.reshape(_B, _S, _D)


# speedup vs baseline: 6.5065x; 2.1674x over previous
import functools

import jax
import jax.numpy as jnp
from jax import lax
from jax.experimental import pallas as pl
from jax.experimental.pallas import tpu as pltpu
from jax.experimental.pallas import tpu_sc as plsc

_D = 1024
_DH = _D // 2
_B, _S = 4, 8192
_NTOK = _B * _S
_ROWS = 2 * _NTOK
_NC, _NS = 2, 16
_NW = _NC * _NS
_TPW = _NTOK // _NW
_CT = 32
_NITER = _TPW // _CT


def _prep_body(gid_tab, gty_tab, row_tab, col_tab, iw, pw, cw,
               gids, gtys, rpos, cpos, h1, h2, i1, i2):
    c = cw[...]
    e = jnp.exp(c - jnp.max(c))
    w = e / jnp.sum(e)
    w0 = w[0, 0]
    w1 = w[0, 1]
    a = w0 * iw[0, 0]
    b = w0 * pw[0, 0]

    def _l2n(x):
        n = jnp.sqrt(jnp.sum(x * x, axis=-1, keepdims=True))
        return x / jnp.maximum(n, 1e-12)

    gin = a * _l2n(gid_tab[...])
    gtn = b * _l2n(gty_tab[...])
    row = w1 * row_tab[...]
    col = w1 * col_tab[...]

    h1[...] = (gin[:, None, None, :_DH] + gtn[None, :, None, :_DH]
               + row[None, None, :, :])
    h2[...] = (gin[:, None, None, _DH:] + gtn[None, :, None, _DH:]
               + col[None, None, :, :])

    base = (gids[...] * 2 + gtys[...]) * 30
    i1[...] = base + rpos[...]
    i2[...] = 600 + base + cpos[...]


_prep = pl.pallas_call(
    _prep_body,
    out_shape=[
        jax.ShapeDtypeStruct((10, 2, 30, _DH), jnp.float32),
        jax.ShapeDtypeStruct((10, 2, 30, _DH), jnp.float32),
        jax.ShapeDtypeStruct((_B, _S), jnp.int32),
        jax.ShapeDtypeStruct((_B, _S), jnp.int32),
    ],
)


def _sc_body(h_hbm, idx1_hbm, idx2_hbm, out_hbm, idx1_all, idx2_all,
             rows0, rows1, sg0, sg1, so0, so1):
    wid = lax.axis_index("s") * _NC + lax.axis_index("c")
    tbase = wid * _TPW

    pltpu.sync_copy(idx1_hbm.at[pl.ds(tbase, _TPW)], idx1_all)
    pltpu.sync_copy(idx2_hbm.at[pl.ds(tbase, _TPW)], idx2_all)

    def _g1(i, rows):
        return (h_hbm.at[idx1_all.at[pl.ds(i * _CT, _CT)]],
                rows.at[:, pl.ds(0, _DH)])

    def _g2(i, rows):
        return (h_hbm.at[idx2_all.at[pl.ds(i * _CT, _CT)]],
                rows.at[:, pl.ds(_DH, _DH)])

    def _dst(i):
        tok = pl.multiple_of(tbase + i * _CT, 8)
        return out_hbm.at[pl.ds(tok, _CT)]

    def _gstart(i, rows, sem):
        pltpu.async_copy(*_g1(i, rows), sem)
        pltpu.async_copy(*_g2(i, rows), sem)

    def _gwait(i, rows, sem):
        pltpu.make_async_copy(*_g1(i, rows), sem).wait()
        pltpu.make_async_copy(*_g2(i, rows), sem).wait()

    def _wstart(i, rows, sem):
        pltpu.async_copy(rows, _dst(i), sem)

    def _wwait(i, rows, sem):
        pltpu.make_async_copy(rows, _dst(i), sem).wait()

    _gstart(0, rows0, sg0)
    _gstart(1, rows1, sg1)

    def step(j, carry):
        i0 = 2 * j
        i1 = i0 + 1
        _gwait(i0, rows0, sg0)
        _wstart(i0, rows0, so0)
        _gwait(i1, rows1, sg1)
        _wstart(i1, rows1, so1)

        @pl.when(j < _NITER // 2 - 1)
        def _():
            _wwait(i0, rows0, so0)
            _gstart(i0 + 2, rows0, sg0)
            _wwait(i1, rows1, so1)
            _gstart(i1 + 2, rows1, sg1)

        return carry

    lax.fori_loop(0, _NITER // 2, step, 0)
    _wwait(_NITER - 2, rows0, so0)
    _wwait(_NITER - 1, rows1, so1)


@functools.cache
def _sc_gather():
    return pl.kernel(
        _sc_body,
        out_type=jax.ShapeDtypeStruct((_NTOK, _D), jnp.float32),
        mesh=plsc.VectorSubcoreMesh(core_axis_name="c", subcore_axis_name="s",
                                    num_cores=_NC, num_subcores=_NS),
        scratch_types=[
            pltpu.VMEM((_TPW,), jnp.int32),
            pltpu.VMEM((_TPW,), jnp.int32),
            pltpu.VMEM((_CT, _D), jnp.float32),
            pltpu.VMEM((_CT, _D), jnp.float32),
            pltpu.SemaphoreType.DMA,
            pltpu.SemaphoreType.DMA,
            pltpu.SemaphoreType.DMA,
            pltpu.SemaphoreType.DMA,
        ],
    )


def kernel(grid_ids, grid_types, row_positions, col_positions,
           grid_id_table, grid_type_table, row_table, col_table,
           input_weight, position_weight, combine_weights):
    gids = grid_ids.astype(jnp.int32)
    gtys = grid_types.astype(jnp.int32)
    rpos = row_positions.astype(jnp.int32)
    cpos = col_positions.astype(jnp.int32)

    h1, h2, i1, i2 = _prep(
        grid_id_table, grid_type_table, row_table, col_table,
        input_weight.reshape(1, 1), position_weight.reshape(1, 1),
        combine_weights.reshape(1, 4), gids, gtys, rpos, cpos)

    htab = jnp.concatenate(
        [h1.reshape(600, _DH), h2.reshape(600, _DH)], axis=0)

    out = _sc_gather()(htab, i1.reshape(-1), i2.reshape(-1))
    return out.reshape(_B, _S, _D)


# speedup vs baseline: 6.6980x; 1.0294x over previous
import functools

import jax
import jax.numpy as jnp
from jax import lax
from jax.experimental import pallas as pl
from jax.experimental.pallas import tpu as pltpu
from jax.experimental.pallas import tpu_sc as plsc

_D = 1024
_DH = _D // 2
_B, _S = 4, 8192
_NTOK = _B * _S
_ROWS = 2 * _NTOK
_NC, _NS = 2, 16
_NW = _NC * _NS
_TPW = _NTOK // _NW
_CT = 16
_NBUF = 4
_NITER = _TPW // _CT


def _prep_body(gid_tab, gty_tab, row_tab, col_tab, iw, pw, cw,
               gids, gtys, rpos, cpos, h1, h2, i1, i2):
    c = cw[...]
    e = jnp.exp(c - jnp.max(c))
    w = e / jnp.sum(e)
    w0 = w[0, 0]
    w1 = w[0, 1]
    a = w0 * iw[0, 0]
    b = w0 * pw[0, 0]

    def _l2n(x):
        n = jnp.sqrt(jnp.sum(x * x, axis=-1, keepdims=True))
        return x / jnp.maximum(n, 1e-12)

    gin = a * _l2n(gid_tab[...])
    gtn = b * _l2n(gty_tab[...])
    row = w1 * row_tab[...]
    col = w1 * col_tab[...]

    h1[...] = (gin[:, None, None, :_DH] + gtn[None, :, None, :_DH]
               + row[None, None, :, :])
    h2[...] = (gin[:, None, None, _DH:] + gtn[None, :, None, _DH:]
               + col[None, None, :, :])

    base = (gids[...] * 2 + gtys[...]) * 30
    i1[...] = base + rpos[...]
    i2[...] = 600 + base + cpos[...]


_prep = pl.pallas_call(
    _prep_body,
    out_shape=[
        jax.ShapeDtypeStruct((10, 2, 30, _DH), jnp.float32),
        jax.ShapeDtypeStruct((10, 2, 30, _DH), jnp.float32),
        jax.ShapeDtypeStruct((_B, _S), jnp.int32),
        jax.ShapeDtypeStruct((_B, _S), jnp.int32),
    ],
)


def _sc_body(h_hbm, idx1_hbm, idx2_hbm, out_hbm, idx1_all, idx2_all,
             rows0, rows1, rows2, rows3,
             sg0, sg1, sg2, sg3, so0, so1, so2, so3):
    wid = lax.axis_index("s") * _NC + lax.axis_index("c")
    tbase = wid * _TPW

    rows = (rows0, rows1, rows2, rows3)
    sg = (sg0, sg1, sg2, sg3)
    so = (so0, so1, so2, so3)

    pltpu.sync_copy(idx1_hbm.at[pl.ds(tbase, _TPW)], idx1_all)
    pltpu.sync_copy(idx2_hbm.at[pl.ds(tbase, _TPW)], idx2_all)

    def _g1(i, buf):
        return (h_hbm.at[idx1_all.at[pl.ds(i * _CT, _CT)]],
                buf.at[:, pl.ds(0, _DH)])

    def _g2(i, buf):
        return (h_hbm.at[idx2_all.at[pl.ds(i * _CT, _CT)]],
                buf.at[:, pl.ds(_DH, _DH)])

    def _dst(i):
        tok = pl.multiple_of(tbase + i * _CT, 8)
        return out_hbm.at[pl.ds(tok, _CT)]

    def _gstart(i, b):
        pltpu.async_copy(*_g1(i, rows[b]), sg[b])
        pltpu.async_copy(*_g2(i, rows[b]), sg[b])

    def _gwait(i, b):
        pltpu.make_async_copy(*_g1(i, rows[b]), sg[b]).wait()
        pltpu.make_async_copy(*_g2(i, rows[b]), sg[b]).wait()

    def _wstart(i, b):
        pltpu.async_copy(rows[b], _dst(i), so[b])

    def _wwait(i, b):
        pltpu.make_async_copy(rows[b], _dst(i), so[b]).wait()

    _gstart(0, 0)
    _gstart(1, 1)

    def step(j, carry):
        for k in range(_NBUF):
            i = 4 * j + k
            _gwait(i, k)
            _wstart(i, k)
            bn = (k + 2) % _NBUF
            if k < 2:
                @pl.when(j > 0)
                def _():
                    _wwait(i - 2, bn)
                _gstart(i + 2, bn)
            else:
                _wwait(i - 2, bn)

                @pl.when(j < _NITER // 4 - 1)
                def _():
                    _gstart(i + 2, bn)
        return carry

    lax.fori_loop(0, _NITER // 4, step, 0)
    _wwait(_NITER - 2, 2)
    _wwait(_NITER - 1, 3)


@functools.cache
def _sc_gather():
    return pl.kernel(
        _sc_body,
        out_type=jax.ShapeDtypeStruct((_NTOK, _D), jnp.float32),
        mesh=plsc.VectorSubcoreMesh(core_axis_name="c", subcore_axis_name="s",
                                    num_cores=_NC, num_subcores=_NS),
        scratch_types=(
            [pltpu.VMEM((_TPW,), jnp.int32)] * 2
            + [pltpu.VMEM((_CT, _D), jnp.float32)] * _NBUF
            + [pltpu.SemaphoreType.DMA] * (2 * _NBUF)
        ),
    )


def kernel(grid_ids, grid_types, row_positions, col_positions,
           grid_id_table, grid_type_table, row_table, col_table,
           input_weight, position_weight, combine_weights):
    gids = grid_ids.astype(jnp.int32)
    gtys = grid_types.astype(jnp.int32)
    rpos = row_positions.astype(jnp.int32)
    cpos = col_positions.astype(jnp.int32)

    h1, h2, i1, i2 = _prep(
        grid_id_table, grid_type_table, row_table, col_table,
        input_weight.reshape(1, 1), position_weight.reshape(1, 1),
        combine_weights.reshape(1, 4), gids, gtys, rpos, cpos)

    htab = jnp.concatenate(
        [h1.reshape(600, _DH), h2.reshape(600, _DH)], axis=0)

    out = _sc_gather()(htab, i1.reshape(-1), i2.reshape(-1))
    return out.reshape(_B, _S, _D)


# speedup vs baseline: 7.0474x; 1.0522x over previous
import functools

import jax
import jax.numpy as jnp
from jax import lax
from jax.experimental import pallas as pl
from jax.experimental.pallas import tpu as pltpu
from jax.experimental.pallas import tpu_sc as plsc

_D = 1024
_DH = _D // 2
_B, _S = 4, 8192
_NTOK = _B * _S
_ROWS = 2 * _NTOK
_NC, _NS = 2, 16
_NW = _NC * _NS
_TPW = _NTOK // _NW
_CT = 16
_NBUF = 4
_NITER = _TPW // _CT


def _prep_body(gid_tab, gty_tab, row_tab, col_tab, iw, pw, cw,
               gids, gtys, rpos, cpos, h, i1, i2):
    c = cw[...]
    e = jnp.exp(c - jnp.max(c))
    w = e / jnp.sum(e)
    w0 = w[0, 0]
    w1 = w[0, 1]
    a = w0 * iw[0, 0]
    b = w0 * pw[0, 0]

    def _l2n(x):
        n = jnp.sqrt(jnp.sum(x * x, axis=-1, keepdims=True))
        return x / jnp.maximum(n, 1e-12)

    gin = a * _l2n(gid_tab[...])
    gtn = b * _l2n(gty_tab[...])
    row = w1 * row_tab[...]
    col = w1 * col_tab[...]

    for g in range(10):
        for y in range(2):
            base = (g * 2 + y) * 30
            glo = (gin[g, :_DH] + gtn[y, :_DH])[None, :]
            ghi = (gin[g, _DH:] + gtn[y, _DH:])[None, :]
            h[pl.ds(base, 30)] = glo + row
            h[pl.ds(600 + base, 30)] = ghi + col

    base = (gids[...] * 2 + gtys[...]) * 30
    i1[...] = base + rpos[...]
    i2[...] = 600 + base + cpos[...]


_prep = pl.pallas_call(
    _prep_body,
    out_shape=[
        jax.ShapeDtypeStruct((1200, _DH), jnp.float32),
        jax.ShapeDtypeStruct((_B, _S), jnp.int32),
        jax.ShapeDtypeStruct((_B, _S), jnp.int32),
    ],
)


def _sc_body(h_hbm, idx1_hbm, idx2_hbm, out_hbm, idx1_all, idx2_all,
             rows0, rows1, rows2, rows3,
             sg0, sg1, sg2, sg3, so0, so1, so2, so3):
    wid = lax.axis_index("s") * _NC + lax.axis_index("c")
    tbase = wid * _TPW

    rows = (rows0, rows1, rows2, rows3)
    sg = (sg0, sg1, sg2, sg3)
    so = (so0, so1, so2, so3)

    brow = wid // (_S // _TPW)
    cbase = (wid % (_S // _TPW)) * _TPW
    pltpu.sync_copy(idx1_hbm.at[brow, pl.ds(cbase, _TPW)], idx1_all)
    pltpu.sync_copy(idx2_hbm.at[brow, pl.ds(cbase, _TPW)], idx2_all)

    def _g1(i, buf):
        return (h_hbm.at[idx1_all.at[pl.ds(i * _CT, _CT)]],
                buf.at[:, pl.ds(0, _DH)])

    def _g2(i, buf):
        return (h_hbm.at[idx2_all.at[pl.ds(i * _CT, _CT)]],
                buf.at[:, pl.ds(_DH, _DH)])

    def _dst(i):
        tok = pl.multiple_of(tbase + i * _CT, 8)
        return out_hbm.at[pl.ds(tok, _CT)]

    def _gstart(i, b):
        pltpu.async_copy(*_g1(i, rows[b]), sg[b])
        pltpu.async_copy(*_g2(i, rows[b]), sg[b])

    def _gwait(i, b):
        pltpu.make_async_copy(*_g1(i, rows[b]), sg[b]).wait()
        pltpu.make_async_copy(*_g2(i, rows[b]), sg[b]).wait()

    def _wstart(i, b):
        pltpu.async_copy(rows[b], _dst(i), so[b])

    def _wwait(i, b):
        pltpu.make_async_copy(rows[b], _dst(i), so[b]).wait()

    _gstart(0, 0)
    _gstart(1, 1)

    def step(j, carry):
        for k in range(_NBUF):
            i = 4 * j + k
            _gwait(i, k)
            _wstart(i, k)
            bn = (k + 2) % _NBUF
            if k < 2:
                @pl.when(j > 0)
                def _():
                    _wwait(i - 2, bn)
                _gstart(i + 2, bn)
            else:
                _wwait(i - 2, bn)

                @pl.when(j < _NITER // 4 - 1)
                def _():
                    _gstart(i + 2, bn)
        return carry

    lax.fori_loop(0, _NITER // 4, step, 0)
    _wwait(_NITER - 2, 2)
    _wwait(_NITER - 1, 3)


@functools.cache
def _sc_gather():
    return pl.kernel(
        _sc_body,
        out_type=jax.ShapeDtypeStruct((_NTOK, _D), jnp.float32),
        mesh=plsc.VectorSubcoreMesh(core_axis_name="c", subcore_axis_name="s",
                                    num_cores=_NC, num_subcores=_NS),
        scratch_types=(
            [pltpu.VMEM((_TPW,), jnp.int32)] * 2
            + [pltpu.VMEM((_CT, _D), jnp.float32)] * _NBUF
            + [pltpu.SemaphoreType.DMA] * (2 * _NBUF)
        ),
    )


def kernel(grid_ids, grid_types, row_positions, col_positions,
           grid_id_table, grid_type_table, row_table, col_table,
           input_weight, position_weight, combine_weights):
    gids = grid_ids.astype(jnp.int32)
    gtys = grid_types.astype(jnp.int32)
    rpos = row_positions.astype(jnp.int32)
    cpos = col_positions.astype(jnp.int32)

    htab, i1, i2 = _prep(
        grid_id_table, grid_type_table, row_table, col_table,
        input_weight.reshape(1, 1), position_weight.reshape(1, 1),
        combine_weights.reshape(1, 4), gids, gtys, rpos, cpos)

    out = _sc_gather()(htab, i1, i2)
    return out.reshape(_B, _S, _D)


# speedup vs baseline: 7.0548x; 1.0011x over previous
import functools

import jax
import jax.numpy as jnp
from jax import lax
from jax.experimental import pallas as pl
from jax.experimental.pallas import tpu as pltpu
from jax.experimental.pallas import tpu_sc as plsc

_D = 1024
_DH = _D // 2
_B, _S = 4, 8192
_NTOK = _B * _S
_ROWS = 2 * _NTOK
_NC, _NS = 2, 16
_NW = _NC * _NS
_TPW = _NTOK // _NW
_CT = 8
_NBUF = 8
_D_INFLIGHT = _NBUF // 2
_NITER = _TPW // _CT


def _prep_body(gid_tab, gty_tab, row_tab, col_tab, iw, pw, cw,
               gids, gtys, rpos, cpos, h, i1, i2):
    c = cw[...]
    e = jnp.exp(c - jnp.max(c))
    w = e / jnp.sum(e)
    w0 = w[0, 0]
    w1 = w[0, 1]
    a = w0 * iw[0, 0]
    b = w0 * pw[0, 0]

    def _l2n(x):
        n = jnp.sqrt(jnp.sum(x * x, axis=-1, keepdims=True))
        return x / jnp.maximum(n, 1e-12)

    gin = a * _l2n(gid_tab[...])
    gtn = b * _l2n(gty_tab[...])
    row = w1 * row_tab[...]
    col = w1 * col_tab[...]

    for g in range(10):
        for y in range(2):
            base = (g * 2 + y) * 30
            glo = (gin[g, :_DH] + gtn[y, :_DH])[None, :]
            ghi = (gin[g, _DH:] + gtn[y, _DH:])[None, :]
            h[pl.ds(base, 30)] = glo + row
            h[pl.ds(600 + base, 30)] = ghi + col

    base = (gids[...] * 2 + gtys[...]) * 30
    i1[...] = base + rpos[...]
    i2[...] = 600 + base + cpos[...]


_prep = pl.pallas_call(
    _prep_body,
    out_shape=[
        jax.ShapeDtypeStruct((1200, _DH), jnp.float32),
        jax.ShapeDtypeStruct((_B, _S), jnp.int32),
        jax.ShapeDtypeStruct((_B, _S), jnp.int32),
    ],
)


def _sc_body(h_hbm, idx1_hbm, idx2_hbm, out_hbm, idx1_all, idx2_all, *scr):
    wid = lax.axis_index("s") * _NC + lax.axis_index("c")
    tbase = wid * _TPW

    rows = scr[:_NBUF]
    sg = scr[_NBUF:2 * _NBUF]
    so = scr[2 * _NBUF:]

    brow = wid // (_S // _TPW)
    cbase = (wid % (_S // _TPW)) * _TPW
    pltpu.sync_copy(idx1_hbm.at[brow, pl.ds(cbase, _TPW)], idx1_all)
    pltpu.sync_copy(idx2_hbm.at[brow, pl.ds(cbase, _TPW)], idx2_all)

    def _g1(i, buf):
        return (h_hbm.at[idx1_all.at[pl.ds(i * _CT, _CT)]],
                buf.at[:, pl.ds(0, _DH)])

    def _g2(i, buf):
        return (h_hbm.at[idx2_all.at[pl.ds(i * _CT, _CT)]],
                buf.at[:, pl.ds(_DH, _DH)])

    def _dst(i):
        tok = pl.multiple_of(tbase + i * _CT, 8)
        return out_hbm.at[pl.ds(tok, _CT)]

    def _gstart(i, b):
        pltpu.async_copy(*_g1(i, rows[b]), sg[b])
        pltpu.async_copy(*_g2(i, rows[b]), sg[b])

    def _gwait(i, b):
        pltpu.make_async_copy(*_g1(i, rows[b]), sg[b]).wait()
        pltpu.make_async_copy(*_g2(i, rows[b]), sg[b]).wait()

    def _wstart(i, b):
        pltpu.async_copy(rows[b], _dst(i), so[b])

    def _wwait(i, b):
        pltpu.make_async_copy(rows[b], _dst(i), so[b]).wait()

    dd = _D_INFLIGHT
    nj = _NITER // _NBUF
    for b in range(dd):
        _gstart(b, b)

    def step(j, carry):
        for k in range(_NBUF):
            i = _NBUF * j + k
            _gwait(i, k)
            _wstart(i, k)
            bn = (k + dd) % _NBUF
            if k < _NBUF - dd:
                @pl.when(j > 0)
                def _():
                    _wwait(i - (_NBUF - dd), bn)
                _gstart(i + dd, bn)
            else:
                _wwait(i - (_NBUF - dd), bn)

                @pl.when(j < nj - 1)
                def _():
                    _gstart(i + dd, bn)
        return carry

    lax.fori_loop(0, nj, step, 0)
    for c in range(_NITER - dd, _NITER):
        _wwait(c, c % _NBUF)


@functools.cache
def _sc_gather():
    return pl.kernel(
        _sc_body,
        out_type=jax.ShapeDtypeStruct((_NTOK, _D), jnp.float32),
        mesh=plsc.VectorSubcoreMesh(core_axis_name="c", subcore_axis_name="s",
                                    num_cores=_NC, num_subcores=_NS),
        scratch_types=(
            [pltpu.VMEM((_TPW,), jnp.int32)] * 2
            + [pltpu.VMEM((_CT, _D), jnp.float32)] * _NBUF
            + [pltpu.SemaphoreType.DMA] * (2 * _NBUF)
        ),
    )


def kernel(grid_ids, grid_types, row_positions, col_positions,
           grid_id_table, grid_type_table, row_table, col_table,
           input_weight, position_weight, combine_weights):
    gids = grid_ids.astype(jnp.int32)
    gtys = grid_types.astype(jnp.int32)
    rpos = row_positions.astype(jnp.int32)
    cpos = col_positions.astype(jnp.int32)

    htab, i1, i2 = _prep(
        grid_id_table, grid_type_table, row_table, col_table,
        input_weight.reshape(1, 1), position_weight.reshape(1, 1),
        combine_weights.reshape(1, 4), gids, gtys, rpos, cpos)

    out = _sc_gather()(htab, i1, i2)
    return out.reshape(_B, _S, _D)
